# baseline (speedup 1.0000x reference)
import jax
import jax.numpy as jnp
from jax import lax
from jax.experimental import pallas as pl
from jax.experimental.pallas import tpu as pltpu
from functools import partial

B, N, S = 8, 4096, 1024
C1, C2 = 128, 256
M1, M2 = 256, 128
NBLK = 2048
NB = N // NBLK
CNT = float(B * N)
EPS = 1e-5


def _pass_a(xyz1_ref, xyz2t_ref, feats2_ref, feats1_ref, w1a_ref, w1b_ref,
            b1_ref, y1_ref, s1_ref, ss1_ref, g_ref, wmat_ref):
    b_i = pl.program_id(0)
    n_i = pl.program_id(1)

    @pl.when(n_i == 0)
    def _():
        g_ref[...] = jnp.dot(w1a_ref[...], feats2_ref[0],
                             preferred_element_type=jnp.float32)

    aq = xyz1_ref[0]
    bb = xyz2t_ref[0]
    a2 = jnp.sum(aq * aq, axis=1)
    b2 = jnp.sum(bb * bb, axis=0)
    abn = jnp.dot(aq.astype(jnp.bfloat16),
                  (-2.0 * bb).astype(jnp.bfloat16),
                  preferred_element_type=jnp.float32)
    t = b2[None, :] + abn

    sels, ws = [], []
    dd = t
    for k in range(3):
        mt = jnp.min(dd, axis=1)
        sel = dd == mt[:, None]
        sels.append(sel)
        m2 = a2 + mt
        ws.append(jnp.minimum(lax.rsqrt(jnp.maximum(m2, 0.0)), 1e8))
        if k < 2:
            dd = jnp.where(sel, jnp.float32(jnp.inf), dd)

    rws = 1.0 / (ws[0] + ws[1] + ws[2])
    wm = (jnp.where(sels[0], (ws[0] * rws)[:, None], 0.0)
          + jnp.where(sels[1], (ws[1] * rws)[:, None], 0.0)
          + jnp.where(sels[2], (ws[2] * rws)[:, None], 0.0))
    wmat_ref[...] = wm

    rs = jnp.sum(wm, axis=1)
    bad = jnp.max(jnp.abs(rs - 1.0))

    @pl.when(bad > 1e-4)
    def _():
        iota1 = lax.broadcasted_iota(jnp.int32, (NBLK, S), 1)
        de = (a2[:, None] + b2[None, :]) + abn
        emasks, ews = [], []
        for k in range(3):
            em2 = jnp.min(de, axis=1)
            esel = de == em2[:, None]
            eidx = jnp.min(jnp.where(esel, iota1, S), axis=1)
            edk = jnp.sqrt(jnp.maximum(em2, 0.0))
            ews.append(1.0 / jnp.maximum(edk, 1e-8))
            emk = iota1 == eidx[:, None]
            emasks.append(emk)
            if k < 2:
                de = jnp.where(emk, jnp.float32(jnp.inf), de)
        ewsum = ews[0] + ews[1] + ews[2]
        wmat_ref[...] = (
            jnp.where(emasks[0], (ews[0] / ewsum)[:, None], 0.0)
            + jnp.where(emasks[1], (ews[1] / ewsum)[:, None], 0.0)
            + jnp.where(emasks[2], (ews[2] / ewsum)[:, None], 0.0))

    y = (lax.dot_general(g_ref[...], wmat_ref[...], (((1,), (1,)), ((), ())),
                         preferred_element_type=jnp.float32)
         + jnp.dot(w1b_ref[...], feats1_ref[0],
                   preferred_element_type=jnp.float32)
         + b1_ref[0][:, None])
    y1_ref[0] = y

    ps = jnp.sum(y, axis=1)
    pss = jnp.sum(y * y, axis=1)
    first = jnp.logical_and(b_i == 0, n_i == 0)

    @pl.when(first)
    def _():
        s1_ref[0] = ps
        ss1_ref[0] = pss

    @pl.when(jnp.logical_not(first))
    def _():
        s1_ref[0] += ps
        ss1_ref[0] += pss


def _pass_b(y1_ref, s1_ref, ss1_ref, g1_ref, be1_ref, w2_ref, b2_ref,
            y2_ref, s2_ref, ss2_ref):
    b_i = pl.program_id(0)
    n_i = pl.program_id(1)
    mean = s1_ref[0] / CNT
    var = ss1_ref[0] / CNT - mean * mean
    rstd = lax.rsqrt(var + EPS)
    scale = rstd * g1_ref[0]
    shift = be1_ref[0] - mean * scale
    z = jnp.maximum(y1_ref[0] * scale[:, None] + shift[:, None], 0.0)
    y = (jnp.dot(w2_ref[...], z, preferred_element_type=jnp.float32)
         + b2_ref[0][:, None])
    y2_ref[0] = y

    ps = jnp.sum(y, axis=1)
    pss = jnp.sum(y * y, axis=1)
    first = jnp.logical_and(b_i == 0, n_i == 0)

    @pl.when(first)
    def _():
        s2_ref[0] = ps
        ss2_ref[0] = pss

    @pl.when(jnp.logical_not(first))
    def _():
        s2_ref[0] += ps
        ss2_ref[0] += pss


def _pass_c(y2_ref, s2_ref, ss2_ref, g2_ref, be2_ref, out_ref):
    mean = s2_ref[0] / CNT
    var = ss2_ref[0] / CNT - mean * mean
    rstd = lax.rsqrt(var + EPS)
    scale = rstd * g2_ref[0]
    shift = be2_ref[0] - mean * scale
    out_ref[0] = jnp.maximum(y2_ref[0] * scale[:, None] + shift[:, None], 0.0)


def kernel(xyz1, xyz2, feats1, feats2, W1, b1, g1, be1, W2, b2, g2, be2):
    xyz2t = jnp.transpose(xyz2, (0, 2, 1))
    w1a = W1[:, :C2]
    w1b = W1[:, C2:]
    b1r = b1.reshape(1, M1)
    g1r = g1.reshape(1, M1)
    be1r = be1.reshape(1, M1)
    b2r = b2.reshape(1, M2)
    g2r = g2.reshape(1, M2)
    be2r = be2.reshape(1, M2)

    y1, s1, ss1 = pl.pallas_call(
        _pass_a,
        grid=(B, NB),
        in_specs=[
            pl.BlockSpec((1, NBLK, 3), lambda b, n: (b, n, 0)),
            pl.BlockSpec((1, 3, S), lambda b, n: (b, 0, 0)),
            pl.BlockSpec((1, C2, S), lambda b, n: (b, 0, 0)),
            pl.BlockSpec((1, C1, NBLK), lambda b, n: (b, 0, n)),
            pl.BlockSpec((M1, C2), lambda b, n: (0, 0)),
            pl.BlockSpec((M1, C1), lambda b, n: (0, 0)),
            pl.BlockSpec((1, M1), lambda b, n: (0, 0)),
        ],
        out_specs=[
            pl.BlockSpec((1, M1, NBLK), lambda b, n: (b, 0, n)),
            pl.BlockSpec((1, M1), lambda b, n: (0, 0)),
            pl.BlockSpec((1, M1), lambda b, n: (0, 0)),
        ],
        out_shape=[
            jax.ShapeDtypeStruct((B, M1, N), jnp.float32),
            jax.ShapeDtypeStruct((1, M1), jnp.float32),
            jax.ShapeDtypeStruct((1, M1), jnp.float32),
        ],
        scratch_shapes=[pltpu.VMEM((M1, S), jnp.float32),
                        pltpu.VMEM((NBLK, S), jnp.float32)],
    )(xyz1, xyz2t, feats2, feats1, w1a, w1b, b1r)

    y2, s2, ss2 = pl.pallas_call(
        _pass_b,
        grid=(B, NB),
        in_specs=[
            pl.BlockSpec((1, M1, NBLK), lambda b, n: (b, 0, n)),
            pl.BlockSpec((1, M1), lambda b, n: (0, 0)),
            pl.BlockSpec((1, M1), lambda b, n: (0, 0)),
            pl.BlockSpec((1, M1), lambda b, n: (0, 0)),
            pl.BlockSpec((1, M1), lambda b, n: (0, 0)),
            pl.BlockSpec((M2, M1), lambda b, n: (0, 0)),
            pl.BlockSpec((1, M2), lambda b, n: (0, 0)),
        ],
        out_specs=[
            pl.BlockSpec((1, M2, NBLK), lambda b, n: (b, 0, n)),
            pl.BlockSpec((1, M2), lambda b, n: (0, 0)),
            pl.BlockSpec((1, M2), lambda b, n: (0, 0)),
        ],
        out_shape=[
            jax.ShapeDtypeStruct((B, M2, N), jnp.float32),
            jax.ShapeDtypeStruct((1, M2), jnp.float32),
            jax.ShapeDtypeStruct((1, M2), jnp.float32),
        ],
    )(y1, s1, ss1, g1r, be1r, W2, b2r)

    out = pl.pallas_call(
        _pass_c,
        grid=(B, NB),
        in_specs=[
            pl.BlockSpec((1, M2, NBLK), lambda b, n: (b, 0, n)),
            pl.BlockSpec((1, M2), lambda b, n: (0, 0)),
            pl.BlockSpec((1, M2), lambda b, n: (0, 0)),
            pl.BlockSpec((1, M2), lambda b, n: (0, 0)),
            pl.BlockSpec((1, M2), lambda b, n: (0, 0)),
        ],
        out_specs=pl.BlockSpec((1, M2, NBLK), lambda b, n: (b, 0, n)),
        out_shape=jax.ShapeDtypeStruct((B, M2, N), jnp.float32),
    )(y2, s2, ss2, g2r, be2r)

    return out
